# baseline (speedup 1.0000x reference)
import functools

import jax
import jax.numpy as jnp
from jax import lax
from jax.experimental import pallas as pl
from jax.experimental.pallas import tpu as pltpu
from jax.experimental.pallas import tpu_sc as plsc

N = 10000
E = 320000
H = 128

NC = 2
NS = 16
LANES = 16
NW = NC * NS
EPT = 10240
EPAD = NW * EPT
K = 64
C = EPT // K
SB = 32
SS = C // SB
NPAD = 10240
RPT = NPAD // NS
FV = H // LANES

_mesh = plsc.VectorSubcoreMesh(
    core_axis_name="c", subcore_axis_name="s", num_cores=NC, num_subcores=NS)


def _zero_stripe(buf, acc_sh, sid):
    zv = jnp.zeros((LANES,), jnp.float32)

    @pl.loop(0, K)
    def _zfill(r):
        for d in range(FV):
            buf[r, pl.ds(d * LANES, LANES)] = zv

    @pl.loop(0, RPT // K)
    def _zcp(i):
        pltpu.sync_copy(buf, acc_sh.at[pl.ds(sid * RPT + i * K, K)])

    rem = RPT % K
    if rem:
        pltpu.sync_copy(
            buf.at[pl.ds(0, rem)],
            acc_sh.at[pl.ds(sid * RPT + (RPT // K) * K, rem)])


def _bcast_rows(w_v, j, buf):
    @pl.loop(0, K // LANES)
    def _grp(g):
        wvec = w_v[j, pl.ds(g * LANES, LANES)]
        for l in range(LANES):
            wb = jnp.full((LANES,), wvec[l], jnp.float32)
            for d in range(FV):
                buf[g * LANES + l, pl.ds(d * LANES, LANES)] = wb


def _scale_rows(w_v, j, buf):
    @pl.loop(0, K // LANES)
    def _grp(g):
        wvec = w_v[j, pl.ds(g * LANES, LANES)]
        for l in range(LANES):
            wb = jnp.full((LANES,), wvec[l], jnp.float32)
            k = g * LANES + l
            for d in range(FV):
                sl = pl.ds(d * LANES, LANES)
                buf[k, sl] = buf[k, sl] * wb


_DEG_OUT = jax.ShapeDtypeStruct((NC, NPAD, H), jnp.float32)
_DEG_SCRATCH = [
    pltpu.VMEM((SB, K), jnp.int32),
    pltpu.VMEM((SB, K), jnp.float32),
    pltpu.VMEM((K, H), jnp.float32),
    pltpu.VMEM((K, H), jnp.float32),
    pltpu.VMEM_SHARED((NPAD, H), jnp.float32),
    pltpu.SemaphoreType.DMA,
    pltpu.SemaphoreType.DMA,
]


def _sc_deg_body(col_hbm, w_hbm, out_hbm, col_v, w_v, b0, b1, acc_sh,
                 ss0, ss1):
    cid = lax.axis_index("c")
    sid = lax.axis_index("s")
    wid = sid * NC + cid

    _zero_stripe(b0, acc_sh, sid)
    plsc.subcore_barrier()

    bufs = (b0, b1)
    sems = (ss0, ss1)

    @pl.loop(0, SS)
    def _super(sc):
        pltpu.sync_copy(col_hbm.at[wid, sc], col_v)
        pltpu.sync_copy(w_hbm.at[wid, sc], w_v)

        @pl.loop(0, SB // 2)
        def _pair(t):
            for par in range(2):
                j = t * 2 + par
                buf, sem = bufs[par], sems[par]

                @pl.when(t > 0)
                def _():
                    pltpu.make_async_copy(
                        buf, acc_sh.at[col_v.at[j - 2]], sem).wait()

                _bcast_rows(w_v, j, buf)
                pltpu.async_copy(buf, acc_sh.at[col_v.at[j]], sem, add=True)

        for par in range(2):
            pltpu.make_async_copy(
                bufs[par], acc_sh.at[col_v.at[SB - 2 + par]],
                sems[par]).wait()

    plsc.subcore_barrier()
    pltpu.sync_copy(acc_sh.at[pl.ds(sid * RPT, RPT)],
                    out_hbm.at[cid, pl.ds(sid * RPT, RPT)])


_sc_deg = pl.kernel(_sc_deg_body, out_type=_DEG_OUT, mesh=_mesh,
                    scratch_types=_DEG_SCRATCH)


_SPMM_OUT = jax.ShapeDtypeStruct((NC, NPAD, H), jnp.float32)
_SPMM_SCRATCH = [
    pltpu.VMEM((SB, K), jnp.int32),
    pltpu.VMEM((SB, K), jnp.int32),
    pltpu.VMEM((SB, K), jnp.float32),
    pltpu.VMEM((K, H), jnp.float32),
    pltpu.VMEM((K, H), jnp.float32),
    pltpu.VMEM_SHARED((NPAD, H), jnp.float32),
    pltpu.SemaphoreType.DMA,
    pltpu.SemaphoreType.DMA,
    pltpu.SemaphoreType.DMA,
    pltpu.SemaphoreType.DMA,
]


def _sc_spmm_body(hs_hbm, row_hbm, col_hbm, w_hbm, out_hbm,
                  row_v, col_v, w_v, b0, b1, acc_sh, sg0, sg1, ss0, ss1):
    cid = lax.axis_index("c")
    sid = lax.axis_index("s")
    wid = sid * NC + cid

    _zero_stripe(b0, acc_sh, sid)
    plsc.subcore_barrier()

    bufs = (b0, b1)
    gsems = (sg0, sg1)
    ssems = (ss0, ss1)

    @pl.loop(0, SS)
    def _super(sc):
        pltpu.sync_copy(row_hbm.at[wid, sc], row_v)
        pltpu.sync_copy(col_hbm.at[wid, sc], col_v)
        pltpu.sync_copy(w_hbm.at[wid, sc], w_v)
        pltpu.async_copy(hs_hbm.at[row_v.at[0]], b0, sg0)

        @pl.loop(0, SB // 2)
        def _pair(t):
            for par in range(2):
                j = t * 2 + par
                buf, gsem, ssem = bufs[par], gsems[par], ssems[par]
                nbuf, ngsem = bufs[1 - par], gsems[1 - par]
                nssem = ssems[1 - par]
                pltpu.make_async_copy(hs_hbm.at[row_v.at[j]], buf,
                                      gsem).wait()

                if par == 0:
                    @pl.when(t > 0)
                    def _():
                        pltpu.make_async_copy(
                            nbuf, acc_sh.at[col_v.at[j - 1]], nssem).wait()
                else:
                    pltpu.make_async_copy(
                        nbuf, acc_sh.at[col_v.at[j - 1]], nssem).wait()

                @pl.when(j < SB - 1)
                def _():
                    pltpu.async_copy(hs_hbm.at[row_v.at[j + 1]], nbuf, ngsem)

                _scale_rows(w_v, j, buf)
                pltpu.async_copy(buf, acc_sh.at[col_v.at[j]], ssem, add=True)

        pltpu.make_async_copy(b1, acc_sh.at[col_v.at[SB - 1]], ss1).wait()

    plsc.subcore_barrier()
    pltpu.sync_copy(acc_sh.at[pl.ds(sid * RPT, RPT)],
                    out_hbm.at[cid, pl.ds(sid * RPT, RPT)])


_sc_spmm = pl.kernel(_sc_spmm_body, out_type=_SPMM_OUT, mesh=_mesh,
                     scratch_types=_SPMM_SCRATCH)


def _dinv16(degp):
    d = degp[0][:N, :LANES] + degp[1][:N, :LANES] + 1.0
    return jnp.where(d > 0, lax.rsqrt(d), 0.0)


def _tc_pro_body(x_ref, w_ref, degp_ref, hs_ref, dinv_ref):
    dinv = _dinv16(degp_ref[...])
    dinv_ref[...] = dinv
    h = jnp.dot(x_ref[...], w_ref[...], preferred_element_type=jnp.float32)
    hs_ref[...] = h * dinv[:, :1]


_tc_pro = pl.pallas_call(
    _tc_pro_body,
    out_shape=(jax.ShapeDtypeStruct((N, H), jnp.float32),
               jax.ShapeDtypeStruct((N, LANES), jnp.float32)),
)


def _tc_epi_body(has_next, p_ref, hs_ref, dinv_ref, b_ref, g_ref, be_ref,
                 *rest):
    if has_next:
        wn_ref, out_ref = rest
    else:
        (out_ref,) = rest
    dinv = dinv_ref[:, :1]
    conv = dinv * (p_ref[0][:N] + p_ref[1][:N] + hs_ref[...]) + b_ref[...]
    a = jnp.maximum(conv, 0.0)
    mean = jnp.mean(a, axis=0, keepdims=True)
    ctr = a - mean
    var = jnp.mean(ctr * ctr, axis=0, keepdims=True)
    y = g_ref[...] * ctr * lax.rsqrt(var + 1e-5) + be_ref[...]
    if has_next:
        out_ref[...] = dinv * jnp.dot(
            y, wn_ref[...], preferred_element_type=jnp.float32)
    else:
        out_ref[...] = y


_tc_mid = pl.pallas_call(
    functools.partial(_tc_epi_body, True),
    out_shape=jax.ShapeDtypeStruct((N, H), jnp.float32),
)

_tc_fin = pl.pallas_call(
    functools.partial(_tc_epi_body, False),
    out_shape=jax.ShapeDtypeStruct((N, H), jnp.float32),
)


def kernel(x, edge_index, edge_weights, W1, b1, g1, be1, W2, b2, g2, be2,
           W3, b3, g3, be3):
    pad = EPAD - E
    zi = jnp.zeros((pad,), edge_index.dtype)
    row = jnp.concatenate([edge_index[0], zi]).reshape(NW, SS, SB, K)
    col = jnp.concatenate([edge_index[1], zi]).reshape(NW, SS, SB, K)
    w = jnp.concatenate([edge_weights, jnp.zeros((pad,), edge_weights.dtype)]
                        ).reshape(NW, SS, SB, K)
    b1, g1, be1 = (v.reshape(1, H) for v in (b1, g1, be1))
    b2, g2, be2 = (v.reshape(1, H) for v in (b2, g2, be2))
    b3, g3, be3 = (v.reshape(1, H) for v in (b3, g3, be3))

    degp = _sc_deg(col, w)
    hs1, dinv = _tc_pro(x, W1, degp)
    p = _sc_spmm(hs1, row, col, w)
    hs2 = _tc_mid(p, hs1, dinv, b1, g1, be1, W2)
    p = _sc_spmm(hs2, row, col, w)
    hs3 = _tc_mid(p, hs2, dinv, b2, g2, be2, W3)
    p = _sc_spmm(hs3, row, col, w)
    return _tc_fin(p, hs3, dinv, b3, g3, be3)


# speedup vs baseline: 1.0271x; 1.0271x over previous
import functools

import jax
import jax.numpy as jnp
from jax import lax
from jax.experimental import pallas as pl
from jax.experimental.pallas import tpu as pltpu
from jax.experimental.pallas import tpu_sc as plsc

N = 10000
E = 320000
H = 128

NC = 2
NS = 16
LANES = 16
NW = NC * NS
EPT = 10240
EPAD = NW * EPT
K = 64
C = EPT // K
SB = 32
SS = C // SB
NPAD = 10240
RPT = NPAD // NS
FV = H // LANES

_mesh = plsc.VectorSubcoreMesh(
    core_axis_name="c", subcore_axis_name="s", num_cores=NC, num_subcores=NS)


def _zero_stripe(buf, acc_sh, sid):
    zv = jnp.zeros((LANES,), jnp.float32)

    @pl.loop(0, K)
    def _zfill(r):
        for d in range(FV):
            buf[r, pl.ds(d * LANES, LANES)] = zv

    @pl.loop(0, RPT // K)
    def _zcp(i):
        pltpu.sync_copy(buf, acc_sh.at[pl.ds(sid * RPT + i * K, K)])

    rem = RPT % K
    if rem:
        pltpu.sync_copy(
            buf.at[pl.ds(0, rem)],
            acc_sh.at[pl.ds(sid * RPT + (RPT // K) * K, rem)])


def _bcast_rows(w_v, j, buf):
    @pl.loop(0, K // LANES)
    def _grp(g):
        wvec = w_v[j, pl.ds(g * LANES, LANES)]
        for l in range(LANES):
            wb = jnp.full((LANES,), wvec[l], jnp.float32)
            for d in range(FV):
                buf[g * LANES + l, pl.ds(d * LANES, LANES)] = wb


def _scale_rows(w_v, j, buf):
    @pl.loop(0, K // LANES)
    def _grp(g):
        wvec = w_v[j, pl.ds(g * LANES, LANES)]
        for l in range(LANES):
            wb = jnp.full((LANES,), wvec[l], jnp.float32)
            k = g * LANES + l
            for d in range(FV):
                sl = pl.ds(d * LANES, LANES)
                buf[k, sl] = buf[k, sl] * wb


_DEG_OUT = jax.ShapeDtypeStruct((NC, NPAD, H), jnp.float32)
_DEG_SCRATCH = [
    pltpu.VMEM((SB, K), jnp.int32),
    pltpu.VMEM((SB, K), jnp.float32),
    pltpu.VMEM((K, H), jnp.float32),
    pltpu.VMEM((K, H), jnp.float32),
    pltpu.VMEM_SHARED((NPAD, H), jnp.float32),
    pltpu.SemaphoreType.DMA,
    pltpu.SemaphoreType.DMA,
]


def _sc_deg_body(col_hbm, w_hbm, out_hbm, col_v, w_v, b0, b1, acc_sh,
                 ss0, ss1):
    cid = lax.axis_index("c")
    sid = lax.axis_index("s")
    wid = sid * NC + cid

    _zero_stripe(b0, acc_sh, sid)
    plsc.subcore_barrier()

    bufs = (b0, b1)
    sems = (ss0, ss1)

    @pl.loop(0, SS)
    def _super(sc):
        pltpu.sync_copy(col_hbm.at[wid, sc], col_v)
        pltpu.sync_copy(w_hbm.at[wid, sc], w_v)

        @pl.loop(0, SB // 2)
        def _pair(t):
            for par in range(2):
                j = t * 2 + par
                buf, sem = bufs[par], sems[par]

                @pl.when(t > 0)
                def _():
                    pltpu.make_async_copy(
                        buf, acc_sh.at[col_v.at[j - 2]], sem).wait()

                _bcast_rows(w_v, j, buf)
                pltpu.async_copy(buf, acc_sh.at[col_v.at[j]], sem, add=True)

        for par in range(2):
            pltpu.make_async_copy(
                bufs[par], acc_sh.at[col_v.at[SB - 2 + par]],
                sems[par]).wait()

    plsc.subcore_barrier()
    pltpu.sync_copy(acc_sh.at[pl.ds(sid * RPT, RPT)],
                    out_hbm.at[cid, pl.ds(sid * RPT, RPT)])


_sc_deg = pl.kernel(_sc_deg_body, out_type=_DEG_OUT, mesh=_mesh,
                    scratch_types=_DEG_SCRATCH)


_SPMM_OUT = jax.ShapeDtypeStruct((NC, NPAD, H), jnp.float32)
_SPMM_SCRATCH = [
    pltpu.VMEM((SB, K), jnp.int32),
    pltpu.VMEM((SB, K), jnp.int32),
    pltpu.VMEM((SB, K), jnp.float32),
    pltpu.VMEM((K, H), jnp.float32),
    pltpu.VMEM((K, H), jnp.float32),
    pltpu.VMEM_SHARED((NPAD, H), jnp.float32),
    pltpu.SemaphoreType.DMA,
    pltpu.SemaphoreType.DMA,
    pltpu.SemaphoreType.DMA,
    pltpu.SemaphoreType.DMA,
]


def _sc_spmm_body(hs_hbm, row_hbm, col_hbm, w_hbm, out_hbm,
                  row_v, col_v, w_v, b0, b1, acc_sh, sg0, sg1, ss0, ss1):
    cid = lax.axis_index("c")
    sid = lax.axis_index("s")
    wid = sid * NC + cid

    _zero_stripe(b0, acc_sh, sid)
    plsc.subcore_barrier()

    bufs = (b0, b1)
    gsems = (sg0, sg1)
    ssems = (ss0, ss1)

    @pl.loop(0, SS)
    def _super(sc):
        pltpu.sync_copy(row_hbm.at[wid, sc], row_v)
        pltpu.sync_copy(col_hbm.at[wid, sc], col_v)
        pltpu.sync_copy(w_hbm.at[wid, sc], w_v)
        pltpu.async_copy(hs_hbm.at[row_v.at[0]], b0, sg0)

        @pl.loop(0, SB // 2)
        def _pair(t):
            for par in range(2):
                j = t * 2 + par
                buf, gsem = bufs[par], gsems[par]
                nbuf, ngsem = bufs[1 - par], gsems[1 - par]
                pltpu.make_async_copy(hs_hbm.at[row_v.at[j]], buf,
                                      gsem).wait()

                @pl.when(j < SB - 1)
                def _():
                    pltpu.async_copy(hs_hbm.at[row_v.at[j + 1]], nbuf, ngsem)

                _scale_rows(w_v, j, buf)
                pltpu.sync_copy(buf, acc_sh.at[col_v.at[j]], add=True)

    plsc.subcore_barrier()
    pltpu.sync_copy(acc_sh.at[pl.ds(sid * RPT, RPT)],
                    out_hbm.at[cid, pl.ds(sid * RPT, RPT)])


_sc_spmm = pl.kernel(_sc_spmm_body, out_type=_SPMM_OUT, mesh=_mesh,
                     scratch_types=_SPMM_SCRATCH)


def _dinv16(degp):
    d = degp[0][:N, :LANES] + degp[1][:N, :LANES] + 1.0
    return jnp.where(d > 0, lax.rsqrt(d), 0.0)


def _tc_pro_body(x_ref, w_ref, degp_ref, hs_ref, dinv_ref):
    dinv = _dinv16(degp_ref[...])
    dinv_ref[...] = dinv
    h = jnp.dot(x_ref[...], w_ref[...], preferred_element_type=jnp.float32)
    hs_ref[...] = h * dinv[:, :1]


_tc_pro = pl.pallas_call(
    _tc_pro_body,
    out_shape=(jax.ShapeDtypeStruct((N, H), jnp.float32),
               jax.ShapeDtypeStruct((N, LANES), jnp.float32)),
)


def _tc_epi_body(has_next, p_ref, hs_ref, dinv_ref, b_ref, g_ref, be_ref,
                 *rest):
    if has_next:
        wn_ref, out_ref = rest
    else:
        (out_ref,) = rest
    dinv = dinv_ref[:, :1]
    conv = dinv * (p_ref[0][:N] + p_ref[1][:N] + hs_ref[...]) + b_ref[...]
    a = jnp.maximum(conv, 0.0)
    mean = jnp.mean(a, axis=0, keepdims=True)
    ctr = a - mean
    var = jnp.mean(ctr * ctr, axis=0, keepdims=True)
    y = g_ref[...] * ctr * lax.rsqrt(var + 1e-5) + be_ref[...]
    if has_next:
        out_ref[...] = dinv * jnp.dot(
            y, wn_ref[...], preferred_element_type=jnp.float32)
    else:
        out_ref[...] = y


_tc_mid = pl.pallas_call(
    functools.partial(_tc_epi_body, True),
    out_shape=jax.ShapeDtypeStruct((N, H), jnp.float32),
)

_tc_fin = pl.pallas_call(
    functools.partial(_tc_epi_body, False),
    out_shape=jax.ShapeDtypeStruct((N, H), jnp.float32),
)


def kernel(x, edge_index, edge_weights, W1, b1, g1, be1, W2, b2, g2, be2,
           W3, b3, g3, be3):
    pad = EPAD - E
    zi = jnp.zeros((pad,), edge_index.dtype)
    row = jnp.concatenate([edge_index[0], zi]).reshape(NW, SS, SB, K)
    col = jnp.concatenate([edge_index[1], zi]).reshape(NW, SS, SB, K)
    w = jnp.concatenate([edge_weights, jnp.zeros((pad,), edge_weights.dtype)]
                        ).reshape(NW, SS, SB, K)
    b1, g1, be1 = (v.reshape(1, H) for v in (b1, g1, be1))
    b2, g2, be2 = (v.reshape(1, H) for v in (b2, g2, be2))
    b3, g3, be3 = (v.reshape(1, H) for v in (b3, g3, be3))

    degp = _sc_deg(col, w)
    hs1, dinv = _tc_pro(x, W1, degp)
    p = _sc_spmm(hs1, row, col, w)
    hs2 = _tc_mid(p, hs1, dinv, b1, g1, be1, W2)
    p = _sc_spmm(hs2, row, col, w)
    hs3 = _tc_mid(p, hs2, dinv, b2, g2, be2, W3)
    p = _sc_spmm(hs3, row, col, w)
    return _tc_fin(p, hs3, dinv, b3, g3, be3)


# speedup vs baseline: 2.7216x; 2.6499x over previous
import functools

import jax
import jax.numpy as jnp
from jax import lax
from jax.experimental import pallas as pl
from jax.experimental.pallas import tpu as pltpu
from jax.experimental.pallas import tpu_sc as plsc

N = 10000
E = 320000
H = 128

NC = 2
NS = 16
LANES = 16
NW = NC * NS
EPT = 10240
EPAD = NW * EPT
K = 64
C = EPT // K
SB = 32
SS = C // SB
NPAD = 10240
RPT = NPAD // NS
FV = H // LANES

_mesh = plsc.VectorSubcoreMesh(
    core_axis_name="c", subcore_axis_name="s", num_cores=NC, num_subcores=NS)


def _zero_stripe(buf, acc_sh, sid):
    zv = jnp.zeros((LANES,), jnp.float32)

    @pl.loop(0, K)
    def _zfill(r):
        for d in range(FV):
            buf[r, pl.ds(d * LANES, LANES)] = zv

    @pl.loop(0, RPT // K)
    def _zcp(i):
        pltpu.sync_copy(buf, acc_sh.at[pl.ds(sid * RPT + i * K, K)])

    rem = RPT % K
    if rem:
        pltpu.sync_copy(
            buf.at[pl.ds(0, rem)],
            acc_sh.at[pl.ds(sid * RPT + (RPT // K) * K, rem)])


def _bcast_rows(w_v, j, buf):
    @pl.loop(0, K // LANES)
    def _grp(g):
        wvec = w_v[j, pl.ds(g * LANES, LANES)]
        for l in range(LANES):
            wb = jnp.full((LANES,), wvec[l], jnp.float32)
            for d in range(FV):
                buf[g * LANES + l, pl.ds(d * LANES, LANES)] = wb


def _scale_rows(w_v, j, buf):
    @pl.loop(0, K // LANES)
    def _grp(g):
        wvec = w_v[j, pl.ds(g * LANES, LANES)]
        for l in range(LANES):
            wb = jnp.full((LANES,), wvec[l], jnp.float32)
            k = g * LANES + l
            for d in range(FV):
                sl = pl.ds(d * LANES, LANES)
                buf[k, sl] = buf[k, sl] * wb


_DEG_OUT = jax.ShapeDtypeStruct((NC, NPAD, H), jnp.float32)
_DEG_SCRATCH = [
    pltpu.VMEM((SB, K), jnp.int32),
    pltpu.VMEM((SB, K), jnp.float32),
    pltpu.VMEM((K, H), jnp.float32),
    pltpu.VMEM((K, H), jnp.float32),
    pltpu.VMEM_SHARED((NPAD, H), jnp.float32),
    pltpu.SemaphoreType.DMA,
    pltpu.SemaphoreType.DMA,
]


def _sc_deg_body(col_hbm, w_hbm, out_hbm, col_v, w_v, b0, b1, acc_sh,
                 ss0, ss1):
    cid = lax.axis_index("c")
    sid = lax.axis_index("s")
    wid = sid * NC + cid

    _zero_stripe(b0, acc_sh, sid)
    plsc.subcore_barrier()

    bufs = (b0, b1)
    sems = (ss0, ss1)

    @pl.loop(0, SS)
    def _super(sc):
        pltpu.sync_copy(col_hbm.at[wid, sc], col_v)
        pltpu.sync_copy(w_hbm.at[wid, sc], w_v)

        @pl.loop(0, SB // 2)
        def _pair(t):
            for par in range(2):
                j = t * 2 + par
                buf, sem = bufs[par], sems[par]

                @pl.when(t > 0)
                def _():
                    pltpu.make_async_copy(
                        buf, acc_sh.at[col_v.at[j - 2]], sem).wait()

                _bcast_rows(w_v, j, buf)
                pltpu.async_copy(buf, acc_sh.at[col_v.at[j]], sem, add=True)

        for par in range(2):
            pltpu.make_async_copy(
                bufs[par], acc_sh.at[col_v.at[SB - 2 + par]],
                sems[par]).wait()

    plsc.subcore_barrier()
    pltpu.sync_copy(acc_sh.at[pl.ds(sid * RPT, RPT)],
                    out_hbm.at[cid, pl.ds(sid * RPT, RPT)])


_sc_deg = pl.kernel(_sc_deg_body, out_type=_DEG_OUT, mesh=_mesh,
                    scratch_types=_DEG_SCRATCH)


_SPMM_OUT = jax.ShapeDtypeStruct((NC, NPAD, H), jnp.float32)
_SPMM_SCRATCH = [
    pltpu.VMEM((SB, K), jnp.int32),
    pltpu.VMEM((SB, K), jnp.int32),
    pltpu.VMEM((SB, K), jnp.float32),
    pltpu.VMEM((K, H), jnp.float32),
    pltpu.VMEM((K, H), jnp.float32),
    pltpu.VMEM_SHARED((NPAD, H), jnp.float32),
    pltpu.SemaphoreType.DMA,
    pltpu.SemaphoreType.DMA,
    pltpu.SemaphoreType.DMA,
    pltpu.SemaphoreType.DMA,
]


def _sc_spmm_body(hs_hbm, row_hbm, col_hbm, w_hbm, out_hbm,
                  row_v, col_v, w_v, b0, b1, acc_sh, sg0, sg1, ss0, ss1):
    cid = lax.axis_index("c")
    sid = lax.axis_index("s")
    wid = sid * NC + cid

    _zero_stripe(b0, acc_sh, sid)
    plsc.subcore_barrier()

    bufs = (b0, b1)
    gsems = (sg0, sg1)
    ssems = (ss0, ss1)

    @pl.loop(0, SS)
    def _super(sc):
        pltpu.sync_copy(row_hbm.at[wid, sc], row_v)
        pltpu.sync_copy(col_hbm.at[wid, sc], col_v)
        pltpu.sync_copy(w_hbm.at[wid, sc], w_v)
        pltpu.async_copy(hs_hbm.at[row_v.at[0]], b0, sg0)

        @pl.loop(0, SB // 2)
        def _pair(t):
            for par in range(2):
                j = t * 2 + par
                buf, gsem = bufs[par], gsems[par]
                nbuf, ngsem = bufs[1 - par], gsems[1 - par]
                pltpu.make_async_copy(hs_hbm.at[row_v.at[j]], buf,
                                      gsem).wait()

                @pl.when(j < SB - 1)
                def _():
                    pltpu.async_copy(hs_hbm.at[row_v.at[j + 1]], nbuf, ngsem)

                _scale_rows(w_v, j, buf)
                pltpu.sync_copy(buf, acc_sh.at[col_v.at[j]], add=True)

    plsc.subcore_barrier()
    pltpu.sync_copy(acc_sh.at[pl.ds(sid * RPT, RPT)],
                    out_hbm.at[cid, pl.ds(sid * RPT, RPT)])


_sc_spmm = pl.kernel(_sc_spmm_body, out_type=_SPMM_OUT, mesh=_mesh,
                     scratch_types=_SPMM_SCRATCH)


def _dinv16(degp):
    d = degp[0][:N, :LANES] + degp[1][:N, :LANES] + 1.0
    return jnp.where(d > 0, lax.rsqrt(d), 0.0)


def _tc_pro_body(x_ref, w_ref, degp_ref, hs_ref, dinv_ref):
    dinv = _dinv16(degp_ref[...])
    dinv_ref[...] = dinv
    h = jnp.dot(x_ref[...], w_ref[...], preferred_element_type=jnp.float32)
    hs_ref[...] = h * dinv[:, :1]


_tc_pro = pl.pallas_call(
    _tc_pro_body,
    out_shape=(jax.ShapeDtypeStruct((N, H), jnp.float32),
               jax.ShapeDtypeStruct((N, LANES), jnp.float32)),
)


def _tc_epi_body(has_next, p_ref, hs_ref, dinv_ref, b_ref, g_ref, be_ref,
                 *rest):
    if has_next:
        wn_ref, out_ref = rest
    else:
        (out_ref,) = rest
    dinv = dinv_ref[:, :1]
    conv = dinv * (p_ref[0][:N] + p_ref[1][:N] + hs_ref[...]) + b_ref[...]
    a = jnp.maximum(conv, 0.0)
    mean = jnp.mean(a, axis=0, keepdims=True)
    ctr = a - mean
    var = jnp.mean(ctr * ctr, axis=0, keepdims=True)
    y = g_ref[...] * ctr * lax.rsqrt(var + 1e-5) + be_ref[...]
    if has_next:
        out_ref[...] = dinv * jnp.dot(
            y, wn_ref[...], preferred_element_type=jnp.float32)
    else:
        out_ref[...] = y


_tc_mid = pl.pallas_call(
    functools.partial(_tc_epi_body, True),
    out_shape=jax.ShapeDtypeStruct((N, H), jnp.float32),
)

_tc_fin = pl.pallas_call(
    functools.partial(_tc_epi_body, False),
    out_shape=jax.ShapeDtypeStruct((N, H), jnp.float32),
)


def kernel(x, edge_index, edge_weights, W1, b1, g1, be1, W2, b2, g2, be2,
           W3, b3, g3, be3):
    pad = EPAD - E
    pi = (jnp.arange(pad, dtype=edge_index.dtype) * 37) % N
    row = jnp.concatenate([edge_index[0], pi]).reshape(NW, SS, SB, K)
    col = jnp.concatenate([edge_index[1], pi]).reshape(NW, SS, SB, K)
    w = jnp.concatenate([edge_weights, jnp.zeros((pad,), edge_weights.dtype)]
                        ).reshape(NW, SS, SB, K)
    b1, g1, be1 = (v.reshape(1, H) for v in (b1, g1, be1))
    b2, g2, be2 = (v.reshape(1, H) for v in (b2, g2, be2))
    b3, g3, be3 = (v.reshape(1, H) for v in (b3, g3, be3))

    degp = _sc_deg(col, w)
    hs1, dinv = _tc_pro(x, W1, degp)
    p = _sc_spmm(hs1, row, col, w)
    hs2 = _tc_mid(p, hs1, dinv, b1, g1, be1, W2)
    p = _sc_spmm(hs2, row, col, w)
    hs3 = _tc_mid(p, hs2, dinv, b2, g2, be2, W3)
    p = _sc_spmm(hs3, row, col, w)
    return _tc_fin(p, hs3, dinv, b3, g3, be3)
